# Initial kernel scaffold; baseline (speedup 1.0000x reference)
import jax
import jax.numpy as jnp
from jax.experimental import pallas as pl


def kernel(x, edge_index, Q_w1, Q_b1, W_w1, W_b1, Q_w2, Q_b2, W_w2, W_b2, G_w, G_b, g, bn_out_gamma, bn_out_beta, bn_gamma, bn_beta):
    raise NotImplementedError("write your pallas kernel here")


# speedup vs baseline: 7.3476x; 7.3476x over previous
import functools

import jax
import jax.numpy as jnp
from jax import lax
from jax.experimental import pallas as pl
from jax.experimental.pallas import tpu as pltpu
from jax.experimental.pallas import tpu_sc as plsc

N = 10000
E = 320000
D = 128
H = 256
OUT = 128

NP = 10240
RB = 1024
NBLK = NP // RB
NTILES = 16
CHUNK = 128
NCH = 160
EPAD = NTILES * NCH * CHUNK
NCH_D = 80
EPAD_D = NTILES * NCH_D * CHUNK
ROWS_PER_TILE = NP // NTILES
EPS = 1e-5


def _tcA_body(x_ref, qw_ref, qb_ref, t_ref):
    y = lax.dot_general(x_ref[...], qw_ref[...], (((1,), (1,)), ((), ())),
                        preferred_element_type=jnp.float32)
    y = jnp.maximum(y + qb_ref[...][None, :], 0.0)
    t_ref[0] = y[:, :D]
    t_ref[1] = y[:, D:]


def _tc_a(x_pad, qw, qb):
    return pl.pallas_call(
        _tcA_body,
        grid=(NBLK,),
        in_specs=[
            pl.BlockSpec((RB, D), lambda i: (i, 0)),
            pl.BlockSpec((H, D), lambda i: (0, 0)),
            pl.BlockSpec((H,), lambda i: (0,)),
        ],
        out_specs=pl.BlockSpec((2, RB, D), lambda i: (0, i, 0)),
        out_shape=jax.ShapeDtypeStruct((2, NP, D), jnp.float32),
    )(x_pad, qw, qb)


def _tcB_body(x_ref, alo_ref, ahi_ref, deg_ref, w_ref, wb_ref,
              qw2_ref, qb2_ref, h1_ref, t2_ref):
    deg = jnp.sum(deg_ref[...], axis=(0, 1))
    inv = 1.0 / jnp.maximum(deg, 1.0)
    alo = alo_ref[0] * inv[:, None]
    ahi = ahi_ref[0] * inv[:, None]
    dn = (((1,), (1,)), ((), ()))
    z = lax.dot_general(x_ref[...], w_ref[:, :D], dn,
                        preferred_element_type=jnp.float32)
    z += lax.dot_general(alo, w_ref[:, D:2 * D], dn,
                         preferred_element_type=jnp.float32)
    z += lax.dot_general(ahi, w_ref[:, 2 * D:], dn,
                         preferred_element_type=jnp.float32)
    z = jnp.maximum(z + wb_ref[...][None, :], 0.0)
    nrm = jnp.sqrt(jnp.sum(z * z, axis=1, keepdims=True))
    h1 = z / jnp.maximum(nrm, 1e-12)
    h1_ref[...] = h1
    t2 = lax.dot_general(h1, qw2_ref[...], dn,
                         preferred_element_type=jnp.float32)
    t2 = jnp.maximum(t2 + qb2_ref[...][None, :], 0.0)
    t2_ref[0] = t2[:, :D]
    t2_ref[1] = t2[:, D:]


def _tc_b(x_pad, agg, deg2, w1, wb1, qw2, qb2):
    return pl.pallas_call(
        _tcB_body,
        grid=(NBLK,),
        in_specs=[
            pl.BlockSpec((RB, D), lambda i: (i, 0)),
            pl.BlockSpec((1, RB, D), lambda i: (0, i, 0)),
            pl.BlockSpec((1, RB, D), lambda i: (1, i, 0)),
            pl.BlockSpec((2, NTILES, RB), lambda i: (0, 0, i)),
            pl.BlockSpec((OUT, D + H), lambda i: (0, 0)),
            pl.BlockSpec((OUT,), lambda i: (0,)),
            pl.BlockSpec((H, OUT), lambda i: (0, 0)),
            pl.BlockSpec((H,), lambda i: (0,)),
        ],
        out_specs=[
            pl.BlockSpec((RB, OUT), lambda i: (i, 0)),
            pl.BlockSpec((2, RB, D), lambda i: (0, i, 0)),
        ],
        out_shape=[
            jax.ShapeDtypeStruct((NP, OUT), jnp.float32),
            jax.ShapeDtypeStruct((2, NP, D), jnp.float32),
        ],
    )(x_pad, agg, agg, deg2, w1, wb1, qw2, qb2)


def _tcD_body(h1_ref, alo_ref, ahi_ref, deg_ref, w_ref, wb_ref,
              gw_ref, gb_ref, h3_ref, acc_ref):
    i = pl.program_id(0)
    deg = jnp.sum(deg_ref[...], axis=(0, 1))
    inv = 1.0 / jnp.maximum(deg, 1.0)
    alo = alo_ref[0] * inv[:, None]
    ahi = ahi_ref[0] * inv[:, None]
    dn = (((1,), (1,)), ((), ()))
    z = lax.dot_general(h1_ref[...], w_ref[:, :OUT], dn,
                        preferred_element_type=jnp.float32)
    z += lax.dot_general(alo, w_ref[:, OUT:OUT + D], dn,
                         preferred_element_type=jnp.float32)
    z += lax.dot_general(ahi, w_ref[:, OUT + D:], dn,
                         preferred_element_type=jnp.float32)
    z = jnp.maximum(z + wb_ref[...][None, :], 0.0)
    nrm = jnp.sqrt(jnp.sum(z * z, axis=1, keepdims=True))
    h2 = z / jnp.maximum(nrm, 1e-12)
    h3 = lax.dot_general(h2, gw_ref[...], dn,
                         preferred_element_type=jnp.float32)
    h3 = jnp.maximum(h3 + gb_ref[...][None, :], 0.0)
    h3_ref[...] = h3
    row = i * RB + lax.broadcasted_iota(jnp.int32, (RB, 1), 0)
    h3m = jnp.where(row < N, h3, 0.0)

    @pl.when(i == 0)
    def _():
        acc_ref[...] = jnp.zeros_like(acc_ref)

    s1 = jnp.sum(h3m, axis=0, keepdims=True)
    s2 = jnp.sum(h3m * h3m, axis=0, keepdims=True)
    acc_ref[...] += jnp.concatenate(
        [s1, s2, jnp.zeros((6, OUT), jnp.float32)], axis=0)


def _tc_d(h1, agg, deg2, w2, wb2, gw, gb):
    return pl.pallas_call(
        _tcD_body,
        grid=(NBLK,),
        in_specs=[
            pl.BlockSpec((RB, OUT), lambda i: (i, 0)),
            pl.BlockSpec((1, RB, D), lambda i: (0, i, 0)),
            pl.BlockSpec((1, RB, D), lambda i: (1, i, 0)),
            pl.BlockSpec((2, NTILES, RB), lambda i: (0, 0, i)),
            pl.BlockSpec((OUT, OUT + H), lambda i: (0, 0)),
            pl.BlockSpec((OUT,), lambda i: (0,)),
            pl.BlockSpec((OUT, OUT), lambda i: (0, 0)),
            pl.BlockSpec((OUT,), lambda i: (0,)),
        ],
        out_specs=[
            pl.BlockSpec((RB, OUT), lambda i: (i, 0)),
            pl.BlockSpec((8, OUT), lambda i: (0, 0)),
        ],
        out_shape=[
            jax.ShapeDtypeStruct((NP, OUT), jnp.float32),
            jax.ShapeDtypeStruct((8, OUT), jnp.float32),
        ],
    )(h1, agg, agg, deg2, w2, wb2, gw, gb)


def _tcE_body(h3_ref, acc_ref, go_ref, bo_ref, g_ref, gn_ref, bn_ref,
              out_ref):
    mu = acc_ref[0, :] * (1.0 / N)
    ex2 = acc_ref[1, :] * (1.0 / N)
    var = ex2 - mu * mu
    inv1 = lax.rsqrt(var + EPS)
    gg = g_ref[0]
    var2 = (gg * gg) * go_ref[...] * go_ref[...] * var * inv1 * inv1
    inv2 = lax.rsqrt(var2 + EPS)
    a = gg * go_ref[...] * gn_ref[...] * inv1 * inv2
    b = bn_ref[...] - a * mu
    out_ref[...] = h3_ref[...] * a[None, :] + b[None, :]


def _tc_e(h3, acc, go, bo, g, gn, bn):
    return pl.pallas_call(
        _tcE_body,
        grid=(NBLK,),
        in_specs=[
            pl.BlockSpec((RB, OUT), lambda i: (i, 0)),
            pl.BlockSpec((8, OUT), lambda i: (0, 0)),
            pl.BlockSpec((OUT,), lambda i: (0,)),
            pl.BlockSpec((OUT,), lambda i: (0,)),
            pl.BlockSpec((1,), lambda i: (0,)),
            pl.BlockSpec((OUT,), lambda i: (0,)),
            pl.BlockSpec((OUT,), lambda i: (0,)),
        ],
        out_specs=pl.BlockSpec((RB, OUT), lambda i: (i, 0)),
        out_shape=jax.ShapeDtypeStruct((NP, OUT), jnp.float32),
    )(h3, acc, go, bo, g, gn, bn)


_Z16 = functools.partial(jnp.zeros, (16,))


def _zero_fill(ref, nrows, ncols16):
    def body(r, _):
        for k in range(ncols16):
            ref[r, pl.ds(k * 16, 16)] = _Z16(jnp.float32)
        return 0
    lax.fori_loop(0, nrows, body, 0, unroll=False)


GRP = 16
NGRP = NCH // GRP


def _sc_agg_body(t_hbm, src_hbm, dst_hbm, agg_hbm,
                 isrc, idst, grow, agg_sp, gsem):
    c = lax.axis_index("c")
    s = lax.axis_index("s")
    base = s * ROWS_PER_TILE

    _zero_fill(grow.at[0], CHUNK, D // 16)
    for k in range(ROWS_PER_TILE // CHUNK):
        pltpu.sync_copy(grow.at[0], agg_sp.at[pl.ds(base + k * CHUNK, CHUNK)])
    plsc.subcore_barrier()

    def gstart(j, b):
        return pltpu.async_copy(t_hbm.at[isrc.at[j]], grow.at[b], gsem)

    def gwait(b):
        pltpu.make_async_copy(t_hbm.at[isrc.at[0]], grow.at[b], gsem).wait()

    def group(gi, _):
        pltpu.sync_copy(src_hbm.at[c, s, pl.ds(gi * GRP, GRP)], isrc)
        pltpu.sync_copy(dst_hbm.at[s, pl.ds(gi * GRP, GRP)], idst)
        gstart(0, 0)

        def pair(p, _):
            j = p * 2
            gwait(0)
            gstart(j + 1, 1)
            pltpu.sync_copy(grow.at[0], agg_sp.at[idst.at[j]], add=True)
            gwait(1)

            @pl.when(j + 2 < GRP)
            def _():
                gstart(j + 2, 0)

            pltpu.sync_copy(grow.at[1], agg_sp.at[idst.at[j + 1]], add=True)
            return 0

        lax.fori_loop(0, GRP // 2, pair, 0, unroll=False)
        return 0

    lax.fori_loop(0, NGRP, group, 0, unroll=False)
    plsc.subcore_barrier()

    pltpu.sync_copy(agg_sp.at[pl.ds(base, ROWS_PER_TILE)],
                    agg_hbm.at[c, pl.ds(base, ROWS_PER_TILE)])


def _sc_pass(t_flat, src_idx, dst_t):
    mesh = plsc.VectorSubcoreMesh(core_axis_name="c", subcore_axis_name="s")
    return pl.kernel(
        _sc_agg_body,
        out_type=jax.ShapeDtypeStruct((2, NP, D), jnp.float32),
        mesh=mesh,
        scratch_types=[
            pltpu.VMEM((GRP, CHUNK), jnp.int32),
            pltpu.VMEM((GRP, CHUNK), jnp.int32),
            pltpu.VMEM((2, CHUNK, D), jnp.float32),
            pltpu.VMEM_SHARED((NP, D), jnp.float32),
            pltpu.SemaphoreType.DMA,
        ],
    )(t_flat, src_idx, dst_t)


def _sc_deg_body(ddeg_hbm, deg_hbm, idx_deg, deg_local):
    c = lax.axis_index("c")
    s = lax.axis_index("s")
    pltpu.sync_copy(ddeg_hbm.at[c, s], idx_deg)

    def dz(r, _):
        deg_local[pl.ds(r * 16, 16)] = jnp.zeros((16,), jnp.float32)
        return 0
    lax.fori_loop(0, NP // 16, dz, 0, unroll=False)

    ones16 = jnp.ones((16,), jnp.float32)

    def dchunk(k, _):
        for m in range(CHUNK // 16):
            v = idx_deg[k, pl.ds(m * 16, 16)]
            plsc.addupdate_scatter(deg_local, [v], ones16)
        return 0
    lax.fori_loop(0, NCH_D, dchunk, 0, unroll=False)
    pltpu.sync_copy(deg_local, deg_hbm.at[c, s])


def _sc_deg(dst_deg):
    mesh = plsc.VectorSubcoreMesh(core_axis_name="c", subcore_axis_name="s")
    return pl.kernel(
        _sc_deg_body,
        out_type=jax.ShapeDtypeStruct((2, NTILES, NP), jnp.float32),
        mesh=mesh,
        scratch_types=[
            pltpu.VMEM((NCH_D, CHUNK), jnp.int32),
            pltpu.VMEM((NP,), jnp.float32),
        ],
        compiler_params=pltpu.CompilerParams(needs_layout_passes=False),
    )(dst_deg)


def kernel(x, edge_index, Q_w1, Q_b1, W_w1, W_b1, Q_w2, Q_b2, W_w2, W_b2,
           G_w, G_b, g, bn_out_gamma, bn_out_beta, bn_gamma, bn_beta):
    src = edge_index[0]
    dst = edge_index[1]

    x_pad = jnp.concatenate(
        [x, jnp.zeros((NP - N, D), jnp.float32)], axis=0)

    pe = EPAD - E
    ar = jnp.arange(pe, dtype=jnp.int32)
    src_p = jnp.concatenate([src, (ar * 7919) % N])
    dst_p = jnp.concatenate([dst, N + (ar % (NP - N))])
    src_idx = jnp.stack([src_p, src_p + NP]).reshape(2, NTILES, NCH, CHUNK)
    dst_t = dst_p.reshape(NTILES, NCH, CHUNK)

    hp = E // 2
    pd = EPAD_D - hp
    ard = jnp.arange(pd, dtype=jnp.int32)
    dpad = N + (ard % (NP - N))
    dst_deg = jnp.stack([
        jnp.concatenate([dst[:hp], dpad]).reshape(NTILES, NCH_D, CHUNK),
        jnp.concatenate([dst[hp:], dpad]).reshape(NTILES, NCH_D, CHUNK),
    ])

    deg2 = _sc_deg(dst_deg)
    t1 = _tc_a(x_pad, Q_w1, Q_b1).reshape(2 * NP, D)
    agg1 = _sc_pass(t1, src_idx, dst_t)
    h1, t2 = _tc_b(x_pad, agg1, deg2, W_w1, W_b1, Q_w2, Q_b2)
    agg2 = _sc_pass(t2.reshape(2 * NP, D), src_idx, dst_t)
    h3, acc = _tc_d(h1, agg2, deg2, W_w2, W_b2, G_w, G_b)
    out = _tc_e(h3, acc, bn_out_gamma, bn_out_beta, g, bn_gamma, bn_beta)
    return out[:N]


# speedup vs baseline: 13.8295x; 1.8822x over previous
import functools

import jax
import jax.numpy as jnp
from jax import lax
from jax.experimental import pallas as pl
from jax.experimental.pallas import tpu as pltpu
from jax.experimental.pallas import tpu_sc as plsc

N = 10000
E = 320000
D = 128
H = 256
OUT = 128

NP = 10240
RB = 1024
NBLK = NP // RB
NTILES = 16
CHUNK = 128
NCH = 160
EPAD = NTILES * NCH * CHUNK
NCH_D = 80
EPAD_D = NTILES * NCH_D * CHUNK
ROWS_PER_TILE = NP // NTILES
EPS = 1e-5


def _tcA_body(x_ref, qw_ref, qb_ref, t_ref):
    y = lax.dot_general(x_ref[...], qw_ref[...], (((1,), (1,)), ((), ())),
                        preferred_element_type=jnp.float32)
    y = jnp.maximum(y + qb_ref[...][None, :], 0.0)
    t_ref[0] = y[:, :D]
    t_ref[1] = y[:, D:]


def _tc_a(x_pad, qw, qb):
    return pl.pallas_call(
        _tcA_body,
        grid=(NBLK,),
        in_specs=[
            pl.BlockSpec((RB, D), lambda i: (i, 0)),
            pl.BlockSpec((H, D), lambda i: (0, 0)),
            pl.BlockSpec((H,), lambda i: (0,)),
        ],
        out_specs=pl.BlockSpec((2, RB, D), lambda i: (0, i, 0)),
        out_shape=jax.ShapeDtypeStruct((2, NP, D), jnp.float32),
    )(x_pad, qw, qb)


def _tcB_body(x_ref, alo_ref, ahi_ref, deg_ref, w_ref, wb_ref,
              qw2_ref, qb2_ref, h1_ref, t2_ref):
    deg = jnp.sum(deg_ref[...], axis=(0, 1))
    inv = 1.0 / jnp.maximum(deg, 1.0)
    alo = alo_ref[0] * inv[:, None]
    ahi = ahi_ref[0] * inv[:, None]
    dn = (((1,), (1,)), ((), ()))
    z = lax.dot_general(x_ref[...], w_ref[:, :D], dn,
                        preferred_element_type=jnp.float32)
    z += lax.dot_general(alo, w_ref[:, D:2 * D], dn,
                         preferred_element_type=jnp.float32)
    z += lax.dot_general(ahi, w_ref[:, 2 * D:], dn,
                         preferred_element_type=jnp.float32)
    z = jnp.maximum(z + wb_ref[...][None, :], 0.0)
    nrm = jnp.sqrt(jnp.sum(z * z, axis=1, keepdims=True))
    h1 = z / jnp.maximum(nrm, 1e-12)
    h1_ref[...] = h1
    t2 = lax.dot_general(h1, qw2_ref[...], dn,
                         preferred_element_type=jnp.float32)
    t2 = jnp.maximum(t2 + qb2_ref[...][None, :], 0.0)
    t2_ref[0] = t2[:, :D]
    t2_ref[1] = t2[:, D:]


def _tc_b(x_pad, agg, deg2, w1, wb1, qw2, qb2):
    return pl.pallas_call(
        _tcB_body,
        grid=(NBLK,),
        in_specs=[
            pl.BlockSpec((RB, D), lambda i: (i, 0)),
            pl.BlockSpec((1, RB, D), lambda i: (0, i, 0)),
            pl.BlockSpec((1, RB, D), lambda i: (1, i, 0)),
            pl.BlockSpec((2, NTILES, RB), lambda i: (0, 0, i)),
            pl.BlockSpec((OUT, D + H), lambda i: (0, 0)),
            pl.BlockSpec((OUT,), lambda i: (0,)),
            pl.BlockSpec((H, OUT), lambda i: (0, 0)),
            pl.BlockSpec((H,), lambda i: (0,)),
        ],
        out_specs=[
            pl.BlockSpec((RB, OUT), lambda i: (i, 0)),
            pl.BlockSpec((2, RB, D), lambda i: (0, i, 0)),
        ],
        out_shape=[
            jax.ShapeDtypeStruct((NP, OUT), jnp.float32),
            jax.ShapeDtypeStruct((2, NP, D), jnp.float32),
        ],
    )(x_pad, agg, agg, deg2, w1, wb1, qw2, qb2)


def _tcD_body(h1_ref, alo_ref, ahi_ref, deg_ref, w_ref, wb_ref,
              gw_ref, gb_ref, h3_ref, acc_ref):
    i = pl.program_id(0)
    deg = jnp.sum(deg_ref[...], axis=(0, 1))
    inv = 1.0 / jnp.maximum(deg, 1.0)
    alo = alo_ref[0] * inv[:, None]
    ahi = ahi_ref[0] * inv[:, None]
    dn = (((1,), (1,)), ((), ()))
    z = lax.dot_general(h1_ref[...], w_ref[:, :OUT], dn,
                        preferred_element_type=jnp.float32)
    z += lax.dot_general(alo, w_ref[:, OUT:OUT + D], dn,
                         preferred_element_type=jnp.float32)
    z += lax.dot_general(ahi, w_ref[:, OUT + D:], dn,
                         preferred_element_type=jnp.float32)
    z = jnp.maximum(z + wb_ref[...][None, :], 0.0)
    nrm = jnp.sqrt(jnp.sum(z * z, axis=1, keepdims=True))
    h2 = z / jnp.maximum(nrm, 1e-12)
    h3 = lax.dot_general(h2, gw_ref[...], dn,
                         preferred_element_type=jnp.float32)
    h3 = jnp.maximum(h3 + gb_ref[...][None, :], 0.0)
    h3_ref[...] = h3
    row = i * RB + lax.broadcasted_iota(jnp.int32, (RB, 1), 0)
    h3m = jnp.where(row < N, h3, 0.0)

    @pl.when(i == 0)
    def _():
        acc_ref[...] = jnp.zeros_like(acc_ref)

    s1 = jnp.sum(h3m, axis=0, keepdims=True)
    s2 = jnp.sum(h3m * h3m, axis=0, keepdims=True)
    acc_ref[...] += jnp.concatenate(
        [s1, s2, jnp.zeros((6, OUT), jnp.float32)], axis=0)


def _tc_d(h1, agg, deg2, w2, wb2, gw, gb):
    return pl.pallas_call(
        _tcD_body,
        grid=(NBLK,),
        in_specs=[
            pl.BlockSpec((RB, OUT), lambda i: (i, 0)),
            pl.BlockSpec((1, RB, D), lambda i: (0, i, 0)),
            pl.BlockSpec((1, RB, D), lambda i: (1, i, 0)),
            pl.BlockSpec((2, NTILES, RB), lambda i: (0, 0, i)),
            pl.BlockSpec((OUT, OUT + H), lambda i: (0, 0)),
            pl.BlockSpec((OUT,), lambda i: (0,)),
            pl.BlockSpec((OUT, OUT), lambda i: (0, 0)),
            pl.BlockSpec((OUT,), lambda i: (0,)),
        ],
        out_specs=[
            pl.BlockSpec((RB, OUT), lambda i: (i, 0)),
            pl.BlockSpec((8, OUT), lambda i: (0, 0)),
        ],
        out_shape=[
            jax.ShapeDtypeStruct((NP, OUT), jnp.float32),
            jax.ShapeDtypeStruct((8, OUT), jnp.float32),
        ],
    )(h1, agg, agg, deg2, w2, wb2, gw, gb)


def _tcE_body(h3_ref, acc_ref, go_ref, bo_ref, g_ref, gn_ref, bn_ref,
              out_ref):
    mu = acc_ref[0, :] * (1.0 / N)
    ex2 = acc_ref[1, :] * (1.0 / N)
    var = ex2 - mu * mu
    inv1 = lax.rsqrt(var + EPS)
    gg = g_ref[0]
    var2 = (gg * gg) * go_ref[...] * go_ref[...] * var * inv1 * inv1
    inv2 = lax.rsqrt(var2 + EPS)
    a = gg * go_ref[...] * gn_ref[...] * inv1 * inv2
    b = bn_ref[...] - a * mu
    out_ref[...] = h3_ref[...] * a[None, :] + b[None, :]


def _tc_e(h3, acc, go, bo, g, gn, bn):
    return pl.pallas_call(
        _tcE_body,
        grid=(NBLK,),
        in_specs=[
            pl.BlockSpec((RB, OUT), lambda i: (i, 0)),
            pl.BlockSpec((8, OUT), lambda i: (0, 0)),
            pl.BlockSpec((OUT,), lambda i: (0,)),
            pl.BlockSpec((OUT,), lambda i: (0,)),
            pl.BlockSpec((1,), lambda i: (0,)),
            pl.BlockSpec((OUT,), lambda i: (0,)),
            pl.BlockSpec((OUT,), lambda i: (0,)),
        ],
        out_specs=pl.BlockSpec((RB, OUT), lambda i: (i, 0)),
        out_shape=jax.ShapeDtypeStruct((NP, OUT), jnp.float32),
    )(h3, acc, go, bo, g, gn, bn)


_Z16 = functools.partial(jnp.zeros, (16,))


def _zero_fill(ref, nrows, ncols16):
    def body(r, _):
        for k in range(ncols16):
            ref[r, pl.ds(k * 16, 16)] = _Z16(jnp.float32)
        return 0
    lax.fori_loop(0, nrows, body, 0, unroll=False)


ECH = 64
NRING = 4
NCH_E = EPAD // (NTILES * ECH)
GRP = 32
NGRP = NCH_E // GRP


def _sc_agg_body(t_hbm, src_hbm, dst_hbm, agg_hbm,
                 isrc, idst, grow, agg_sp, gsem):
    c = lax.axis_index("c")
    s = lax.axis_index("s")
    base = s * ROWS_PER_TILE

    _zero_fill(grow.at[0], ECH, D // 16)
    _zero_fill(grow.at[1], ECH, D // 16)
    for k in range(ROWS_PER_TILE // (2 * ECH)):
        pltpu.sync_copy(grow.at[0], agg_sp.at[pl.ds(base + k * 2 * ECH, ECH)])
        pltpu.sync_copy(grow.at[1],
                        agg_sp.at[pl.ds(base + k * 2 * ECH + ECH, ECH)])
    plsc.subcore_barrier()

    def gstart(j, b):
        return pltpu.async_copy(t_hbm.at[isrc.at[j]], grow.at[b], gsem)

    def gwait(b):
        pltpu.make_async_copy(t_hbm.at[isrc.at[0]], grow.at[b], gsem).wait()

    def group(gi, _):
        pltpu.sync_copy(src_hbm.at[c, s, pl.ds(gi * GRP, GRP)], isrc)
        pltpu.sync_copy(dst_hbm.at[s, pl.ds(gi * GRP, GRP)], idst)
        for b in range(NRING - 1):
            gstart(b, b)

        def quad(q, _):
            j0 = q * NRING
            for b in range(NRING):
                j = j0 + b
                gwait(b)

                @pl.when(j + NRING - 1 < GRP)
                def _():
                    gstart(j + NRING - 1, (b + NRING - 1) % NRING)

                pltpu.sync_copy(grow.at[b], agg_sp.at[idst.at[j]], add=True)
            return 0

        lax.fori_loop(0, GRP // NRING, quad, 0, unroll=False)
        return 0

    lax.fori_loop(0, NGRP, group, 0, unroll=False)
    plsc.subcore_barrier()

    pltpu.sync_copy(agg_sp.at[pl.ds(base, ROWS_PER_TILE)],
                    agg_hbm.at[c, pl.ds(base, ROWS_PER_TILE)])


def _sc_pass(t_flat, src_idx, dst_t):
    mesh = plsc.VectorSubcoreMesh(core_axis_name="c", subcore_axis_name="s")
    return pl.kernel(
        _sc_agg_body,
        out_type=jax.ShapeDtypeStruct((2, NP, D), jnp.float32),
        mesh=mesh,
        scratch_types=[
            pltpu.VMEM((GRP, ECH), jnp.int32),
            pltpu.VMEM((GRP, ECH), jnp.int32),
            pltpu.VMEM((NRING, ECH, D), jnp.float32),
            pltpu.VMEM_SHARED((NP, D), jnp.float32),
            pltpu.SemaphoreType.DMA,
        ],
    )(t_flat, src_idx, dst_t)


def _sc_deg_body(ddeg_hbm, deg_hbm, idx_deg, deg_local):
    c = lax.axis_index("c")
    s = lax.axis_index("s")
    pltpu.sync_copy(ddeg_hbm.at[c, s], idx_deg)

    def dz(r, _):
        deg_local[pl.ds(r * 16, 16)] = jnp.zeros((16,), jnp.float32)
        return 0
    lax.fori_loop(0, NP // 16, dz, 0, unroll=False)

    ones16 = jnp.ones((16,), jnp.float32)

    def dchunk(k, _):
        for m in range(CHUNK // 16):
            v = idx_deg[k, pl.ds(m * 16, 16)]
            plsc.addupdate_scatter(deg_local, [v], ones16)
        return 0
    lax.fori_loop(0, NCH_D, dchunk, 0, unroll=False)
    pltpu.sync_copy(deg_local, deg_hbm.at[c, s])


def _sc_deg(dst_deg):
    mesh = plsc.VectorSubcoreMesh(core_axis_name="c", subcore_axis_name="s")
    return pl.kernel(
        _sc_deg_body,
        out_type=jax.ShapeDtypeStruct((2, NTILES, NP), jnp.float32),
        mesh=mesh,
        scratch_types=[
            pltpu.VMEM((NCH_D, CHUNK), jnp.int32),
            pltpu.VMEM((NP,), jnp.float32),
        ],
        compiler_params=pltpu.CompilerParams(needs_layout_passes=False),
    )(dst_deg)


def kernel(x, edge_index, Q_w1, Q_b1, W_w1, W_b1, Q_w2, Q_b2, W_w2, W_b2,
           G_w, G_b, g, bn_out_gamma, bn_out_beta, bn_gamma, bn_beta):
    src = edge_index[0]
    dst = edge_index[1]

    x_pad = jnp.concatenate(
        [x, jnp.zeros((NP - N, D), jnp.float32)], axis=0)

    pe = EPAD - E
    ar = jnp.arange(pe, dtype=jnp.int32)
    src_p = jnp.concatenate([src, (ar * 7919) % N])
    dst_p = jnp.concatenate([dst, N + (ar % (NP - N))])
    src_idx = jnp.stack([src_p, src_p + NP]).reshape(2, NTILES, NCH_E, ECH)
    dst_t = dst_p.reshape(NTILES, NCH_E, ECH)

    hp = E // 2
    pd = EPAD_D - hp
    ard = jnp.arange(pd, dtype=jnp.int32)
    dpad = N + (ard % (NP - N))
    dst_deg = jnp.stack([
        jnp.concatenate([dst[:hp], dpad]).reshape(NTILES, NCH_D, CHUNK),
        jnp.concatenate([dst[hp:], dpad]).reshape(NTILES, NCH_D, CHUNK),
    ])

    deg2 = _sc_deg(dst_deg)
    t1 = _tc_a(x_pad, Q_w1, Q_b1).reshape(2 * NP, D)
    agg1 = _sc_pass(t1, src_idx, dst_t)
    h1, t2 = _tc_b(x_pad, agg1, deg2, W_w1, W_b1, Q_w2, Q_b2)
    agg2 = _sc_pass(t2.reshape(2 * NP, D), src_idx, dst_t)
    h3, acc = _tc_d(h1, agg2, deg2, W_w2, W_b2, G_w, G_b)
    out = _tc_e(h3, acc, bn_out_gamma, bn_out_beta, g, bn_gamma, bn_beta)
    return out[:N]
